# Initial kernel scaffold; baseline (speedup 1.0000x reference)
import functools

import jax
import jax.numpy as jnp
from jax.experimental import pallas as pl
from jax.experimental.pallas import tpu as pltpu

_BI = 8


def _body(x_ref, tbl_ref, out_ref, rt_ref, *, seq_len, max_len):
    @pl.when(pl.program_id(0) == 0)
    def _():
        rt_ref[:] = jnp.flip(tbl_ref[:], axis=0)

    ib = pl.program_id(0)
    xv = x_ref[0]
    for r in range(_BI):
        i = ib * _BI + r
        out_ref[r] = xv + rt_ref[pl.ds(max_len - i, seq_len), :]


def kernel(x, rel_pos_embeddings):
    _, seq_len, hidden = x.shape
    n_rows = rel_pos_embeddings.shape[0]
    max_len = (n_rows - 1) // 2
    pad_front = (-n_rows) % 8
    padded = pad_front + n_rows
    tbl_p = jnp.pad(rel_pos_embeddings, ((pad_front, 0), (0, 0)))

    body = functools.partial(_body, seq_len=seq_len, max_len=max_len)
    return pl.pallas_call(
        body,
        grid=(seq_len // _BI,),
        in_specs=[
            pl.BlockSpec((1, seq_len, hidden), lambda ib: (0, 0, 0)),
            pl.BlockSpec((padded, hidden), lambda ib: (0, 0)),
        ],
        out_specs=pl.BlockSpec((_BI, seq_len, hidden), lambda ib: (ib, 0, 0)),
        out_shape=jax.ShapeDtypeStruct((seq_len, seq_len, hidden), jnp.float32),
        scratch_shapes=[pltpu.VMEM((padded, hidden), jnp.float32)],
    )(x, tbl_p)


# speedup vs baseline: 21.9068x; 21.9068x over previous
import functools

import jax
import jax.numpy as jnp
from jax.experimental import pallas as pl
from jax.experimental.pallas import tpu as pltpu

_BI = 8


def _body(x_ref, rt_ref, out_ref, *, seq_len, max_len):
    ib = pl.program_id(0)
    xv = x_ref[0]
    for r in range(_BI):
        i = ib * _BI + r
        out_ref[r] = xv + rt_ref[pl.ds(max_len - i, seq_len), :]


def kernel(x, rel_pos_embeddings):
    _, seq_len, hidden = x.shape
    n_rows = rel_pos_embeddings.shape[0]
    max_len = (n_rows - 1) // 2
    pad_front = (-n_rows) % 8
    padded = pad_front + n_rows
    rt = jnp.pad(jnp.flip(rel_pos_embeddings, axis=0), ((0, pad_front), (0, 0)))

    body = functools.partial(_body, seq_len=seq_len, max_len=max_len)
    return pl.pallas_call(
        body,
        grid=(seq_len // _BI,),
        in_specs=[
            pl.BlockSpec((1, seq_len, hidden), lambda ib: (0, 0, 0)),
            pl.BlockSpec((padded, hidden), lambda ib: (0, 0)),
        ],
        out_specs=pl.BlockSpec((_BI, seq_len, hidden), lambda ib: (ib, 0, 0)),
        out_shape=jax.ShapeDtypeStruct((seq_len, seq_len, hidden), jnp.float32),
    )(x, rt)


# speedup vs baseline: 21.9165x; 1.0004x over previous
import functools

import jax
import jax.numpy as jnp
from jax.experimental import pallas as pl
from jax.experimental.pallas import tpu as pltpu

_BI = 16


def _body(x_ref, rt_ref, out_ref, *, seq_len, max_len):
    ib = pl.program_id(0)
    xv = x_ref[0]
    for r in range(_BI):
        i = ib * _BI + r
        out_ref[r] = xv + rt_ref[pl.ds(max_len - i, seq_len), :]


def kernel(x, rel_pos_embeddings):
    _, seq_len, hidden = x.shape
    n_rows = rel_pos_embeddings.shape[0]
    max_len = (n_rows - 1) // 2
    pad_front = (-n_rows) % 8
    padded = pad_front + n_rows
    rt = jnp.pad(jnp.flip(rel_pos_embeddings, axis=0), ((0, pad_front), (0, 0)))

    body = functools.partial(_body, seq_len=seq_len, max_len=max_len)
    return pl.pallas_call(
        body,
        grid=(seq_len // _BI,),
        in_specs=[
            pl.BlockSpec((1, seq_len, hidden), lambda ib: (0, 0, 0)),
            pl.BlockSpec((padded, hidden), lambda ib: (0, 0)),
        ],
        out_specs=pl.BlockSpec((_BI, seq_len, hidden), lambda ib: (ib, 0, 0)),
        out_shape=jax.ShapeDtypeStruct((seq_len, seq_len, hidden), jnp.float32),
    )(x, rt)
